# Initial kernel scaffold; baseline (speedup 1.0000x reference)
import jax
import jax.numpy as jnp
from jax.experimental import pallas as pl


def kernel(raw_x, adj, W_att, a_att, W_res):
    raise NotImplementedError("write your pallas kernel here")


# speedup vs baseline: 1.1837x; 1.1837x over previous
import jax
import jax.numpy as jnp
from jax.experimental import pallas as pl
from jax.experimental.pallas import tpu as pltpu

N = 10000
NFEAT = 128
NCLASS = 16
ALPHA = 0.2

BM = 256
BN = 1024
NI = pl.cdiv(N, BM)
NJ = pl.cdiv(N, BN)
NPAD = NI * BM


def _prologue_body(x_ref, watt_ref, a_ref, wres_ref,
                   wh_ref, res_ref, src_ref, dst_ref):
    i = pl.program_id(0)
    x = x_ref[:, :]
    rid = i * BM + jax.lax.broadcasted_iota(jnp.int32, (BM, 1), 0)
    valid = rid < N

    wh = jnp.dot(x, watt_ref[:, :], preferred_element_type=jnp.float32)
    wh = jnp.where(valid, wh, 0.0)
    wh_ref[:, :] = wh

    res = jnp.dot(x, wres_ref[:, :], preferred_element_type=jnp.float32)
    res_ref[:, :] = jnp.where(valid, res, 0.0)

    a = a_ref[:, :]
    src_ref[:, :] = jnp.sum(wh * a[:, :NCLASS], axis=1, keepdims=True)

    dst_row = jax.lax.dot_general(
        a[:, NCLASS:], wh, (((1,), (1,)), ((), ())),
        preferred_element_type=jnp.float32)
    vrow = (i * BM + jax.lax.broadcasted_iota(jnp.int32, (1, BM), 1)) < N
    dst_ref[:, :] = jnp.where(vrow, dst_row, -1e30)


def _main_body(adj_ref, wh_ref, src_ref, dst_ref, res_ref,
               out_ref, acc_ref, l_ref, c_ref):
    j = pl.program_id(1)

    @pl.when(j == 0)
    def _init():
        dmax = jnp.max(dst_ref[:, :])
        c = src_ref[:, :] + dmax
        c_ref[:, :] = jnp.where(c > 0, c, ALPHA * c)
        acc_ref[:, :] = jnp.zeros_like(acc_ref)
        l_ref[:, :] = jnp.zeros_like(l_ref)

    d = dst_ref[:, pl.ds(j * BN, BN)]
    e = src_ref[:, :] + d
    e = jnp.where(e > 0, e, ALPHA * e)
    s = jnp.where(adj_ref[:, :] > 0, e, -9e15)
    p = jnp.exp(s - c_ref[:, :])
    acc_ref[:, :] += jnp.dot(p, wh_ref[:, :],
                             preferred_element_type=jnp.float32)
    l_ref[:, :] += jnp.sum(p, axis=1, keepdims=True)

    @pl.when(j == NJ - 1)
    def _fin():
        h = acc_ref[:, :] / l_ref[:, :]
        h = jnp.where(h > 0, h, jnp.exp(h) - 1.0)
        o = h + res_ref[:, :]
        mx = jnp.max(o, axis=1, keepdims=True)
        lse = mx + jnp.log(jnp.sum(jnp.exp(o - mx), axis=1, keepdims=True))
        out_ref[:, :] = o - lse


@jax.jit
def kernel(raw_x, adj, W_att, a_att, W_res):
    a_row = a_att.reshape(1, 2 * NCLASS)

    wh, res, src, dst = pl.pallas_call(
        _prologue_body,
        grid=(NI,),
        in_specs=[
            pl.BlockSpec((BM, NFEAT), lambda i: (i, 0)),
            pl.BlockSpec((NFEAT, NCLASS), lambda i: (0, 0)),
            pl.BlockSpec((1, 2 * NCLASS), lambda i: (0, 0)),
            pl.BlockSpec((NFEAT, NCLASS), lambda i: (0, 0)),
        ],
        out_specs=[
            pl.BlockSpec((BM, NCLASS), lambda i: (i, 0)),
            pl.BlockSpec((BM, NCLASS), lambda i: (i, 0)),
            pl.BlockSpec((BM, 1), lambda i: (i, 0)),
            pl.BlockSpec((1, BM), lambda i: (0, i)),
        ],
        out_shape=[
            jax.ShapeDtypeStruct((NPAD, NCLASS), jnp.float32),
            jax.ShapeDtypeStruct((NPAD, NCLASS), jnp.float32),
            jax.ShapeDtypeStruct((NPAD, 1), jnp.float32),
            jax.ShapeDtypeStruct((1, NPAD), jnp.float32),
        ],
    )(raw_x, W_att, a_row, W_res)

    out = pl.pallas_call(
        _main_body,
        grid=(NI, NJ),
        in_specs=[
            pl.BlockSpec((BM, BN), lambda i, j: (i, j)),
            pl.BlockSpec((BN, NCLASS), lambda i, j: (j, 0)),
            pl.BlockSpec((BM, 1), lambda i, j: (i, 0)),
            pl.BlockSpec((1, NPAD), lambda i, j: (0, 0)),
            pl.BlockSpec((BM, NCLASS), lambda i, j: (i, 0)),
        ],
        out_specs=pl.BlockSpec((BM, NCLASS), lambda i, j: (i, 0)),
        out_shape=jax.ShapeDtypeStruct((N, NCLASS), jnp.float32),
        scratch_shapes=[
            pltpu.VMEM((BM, NCLASS), jnp.float32),
            pltpu.VMEM((BM, 1), jnp.float32),
            pltpu.VMEM((BM, 1), jnp.float32),
        ],
        compiler_params=pltpu.CompilerParams(
            dimension_semantics=("parallel", "arbitrary"),
        ),
    )(adj, wh, src, dst, res)

    return out


# speedup vs baseline: 1.2497x; 1.0557x over previous
import jax
import jax.numpy as jnp
from jax.experimental import pallas as pl
from jax.experimental.pallas import tpu as pltpu

N = 10000
NFEAT = 128
NCLASS = 16
ALPHA = 0.2
LOG2E = 1.4426950408889634

BM = 256
BN = 1024
NI = pl.cdiv(N, BM)
NJ = pl.cdiv(N, BN)
NPAD = NI * BM


def _prologue_body(x_ref, watt_ref, a_ref, wres_ref,
                   wh_ref, res_ref, src_ref, d_ref, w_ref):
    i = pl.program_id(0)
    x = x_ref[:, :]
    rid = i * BM + jax.lax.broadcasted_iota(jnp.int32, (BM, 1), 0)
    valid = rid < N

    wh = jnp.dot(x, watt_ref[:, :], preferred_element_type=jnp.float32)
    wh = jnp.where(valid, wh, 0.0)
    wh_ref[:, :NCLASS] = wh
    wh_ref[:, NCLASS:] = jnp.ones((BM, 1), jnp.float32)

    res = jnp.dot(x, wres_ref[:, :], preferred_element_type=jnp.float32)
    res_ref[:, :] = jnp.where(valid, res, 0.0)

    a = a_ref[:, :]
    src_ref[:, :] = jnp.sum(wh * a[:, :NCLASS], axis=1, keepdims=True)

    dst_row = jax.lax.dot_general(
        a[:, NCLASS:], wh, (((1,), (1,)), ((), ())),
        preferred_element_type=jnp.float32)
    vrow = (i * BM + jax.lax.broadcasted_iota(jnp.int32, (1, BM), 1)) < N
    dst_row = jnp.where(vrow, dst_row, -1e30)
    d_ref[:, :] = LOG2E * dst_row
    w_ref[:, :] = (ALPHA * LOG2E) * dst_row


def _main_body(adj_ref, wh_ref, src_ref, d_ref, w_ref, res_ref,
               out_ref, acc_ref, u_ref, v_ref):
    j = pl.program_id(1)

    @pl.when(j == 0)
    def _init():
        dmax2 = jnp.max(d_ref[:, :])
        src2 = LOG2E * src_ref[:, :]
        etop = src2 + dmax2
        c2 = jnp.maximum(etop, ALPHA * etop)
        u_ref[:, :] = src2 - c2
        v_ref[:, :] = ALPHA * src2 - c2
        acc_ref[:, :] = jnp.zeros_like(acc_ref)

    d = d_ref[:, pl.ds(j * BN, BN)]
    w = w_ref[:, pl.ds(j * BN, BN)]
    s2 = jnp.maximum(u_ref[:, :] + d, v_ref[:, :] + w)
    p = jnp.where(adj_ref[:, :] > 0, jnp.exp2(s2), 0.0)
    acc_ref[:, :] += jnp.dot(p, wh_ref[:, :],
                             preferred_element_type=jnp.float32)

    @pl.when(j == NJ - 1)
    def _fin():
        h = acc_ref[:, :NCLASS] / acc_ref[:, NCLASS:]
        h = jnp.where(h > 0, h, jnp.exp(h) - 1.0)
        o = h + res_ref[:, :]
        mx = jnp.max(o, axis=1, keepdims=True)
        lse = mx + jnp.log(jnp.sum(jnp.exp(o - mx), axis=1, keepdims=True))
        out_ref[:, :] = o - lse


@jax.jit
def kernel(raw_x, adj, W_att, a_att, W_res):
    a_row = a_att.reshape(1, 2 * NCLASS)

    wh, res, src, d2, w2 = pl.pallas_call(
        _prologue_body,
        grid=(NI,),
        in_specs=[
            pl.BlockSpec((BM, NFEAT), lambda i: (i, 0)),
            pl.BlockSpec((NFEAT, NCLASS), lambda i: (0, 0)),
            pl.BlockSpec((1, 2 * NCLASS), lambda i: (0, 0)),
            pl.BlockSpec((NFEAT, NCLASS), lambda i: (0, 0)),
        ],
        out_specs=[
            pl.BlockSpec((BM, NCLASS + 1), lambda i: (i, 0)),
            pl.BlockSpec((BM, NCLASS), lambda i: (i, 0)),
            pl.BlockSpec((BM, 1), lambda i: (i, 0)),
            pl.BlockSpec((1, BM), lambda i: (0, i)),
            pl.BlockSpec((1, BM), lambda i: (0, i)),
        ],
        out_shape=[
            jax.ShapeDtypeStruct((NPAD, NCLASS + 1), jnp.float32),
            jax.ShapeDtypeStruct((NPAD, NCLASS), jnp.float32),
            jax.ShapeDtypeStruct((NPAD, 1), jnp.float32),
            jax.ShapeDtypeStruct((1, NPAD), jnp.float32),
            jax.ShapeDtypeStruct((1, NPAD), jnp.float32),
        ],
    )(raw_x, W_att, a_row, W_res)

    out = pl.pallas_call(
        _main_body,
        grid=(NI, NJ),
        in_specs=[
            pl.BlockSpec((BM, BN), lambda i, j: (i, j)),
            pl.BlockSpec((BN, NCLASS + 1), lambda i, j: (j, 0)),
            pl.BlockSpec((BM, 1), lambda i, j: (i, 0)),
            pl.BlockSpec((1, NPAD), lambda i, j: (0, 0)),
            pl.BlockSpec((1, NPAD), lambda i, j: (0, 0)),
            pl.BlockSpec((BM, NCLASS), lambda i, j: (i, 0)),
        ],
        out_specs=pl.BlockSpec((BM, NCLASS), lambda i, j: (i, 0)),
        out_shape=jax.ShapeDtypeStruct((N, NCLASS), jnp.float32),
        scratch_shapes=[
            pltpu.VMEM((BM, NCLASS + 1), jnp.float32),
            pltpu.VMEM((BM, 1), jnp.float32),
            pltpu.VMEM((BM, 1), jnp.float32),
        ],
        compiler_params=pltpu.CompilerParams(
            dimension_semantics=("parallel", "arbitrary"),
        ),
    )(adj, wh, src, d2, w2, res)

    return out


# speedup vs baseline: 2.7372x; 2.1903x over previous
import jax
import jax.numpy as jnp
from jax.experimental import pallas as pl
from jax.experimental.pallas import tpu as pltpu

N = 10000
NFEAT = 128
NCLASS = 16
ALPHA = 0.2
LOG2E = 1.4426950408889634

BM = 128
NI = pl.cdiv(N, BM)
BMP = 512
NIP = pl.cdiv(N, BMP)


def _prologue_body(x_ref, watt_ref, a_ref, wres_ref,
                   wh_ref, res_ref, src_ref, d_ref, w_ref):
    x = x_ref[:, :]
    wh = jnp.dot(x, watt_ref[:, :], preferred_element_type=jnp.float32)
    wh_ref[:, :NCLASS] = wh
    wh_ref[:, NCLASS:] = jnp.ones((BMP, 1), jnp.float32)

    res_ref[:, :] = jnp.dot(x, wres_ref[:, :],
                            preferred_element_type=jnp.float32)

    a = a_ref[:, :]
    src_ref[:, :] = jnp.sum(wh * a[:, :NCLASS], axis=1, keepdims=True)

    dst_row = jax.lax.dot_general(
        a[:, NCLASS:], wh, (((1,), (1,)), ((), ())),
        preferred_element_type=jnp.float32)
    d_ref[:, :] = LOG2E * dst_row
    w_ref[:, :] = (ALPHA * LOG2E) * dst_row


def _main_body(adj_ref, wh_ref, src_ref, d_ref, w_ref, res_ref, out_ref):
    dmax2 = jnp.max(d_ref[:, :])
    src2 = LOG2E * src_ref[:, :]
    etop = src2 + dmax2
    c2 = jnp.maximum(etop, ALPHA * etop)
    u = src2 - c2
    v = ALPHA * src2 - c2

    s2 = jnp.maximum(u + d_ref[:, :], v + w_ref[:, :])
    p = adj_ref[:, :] * jnp.exp2(s2)
    acc = jnp.dot(p, wh_ref[:, :], preferred_element_type=jnp.float32)

    h = acc[:, :NCLASS] / acc[:, NCLASS:]
    h = jnp.where(h > 0, h, jnp.exp(h) - 1.0)
    o = h + res_ref[:, :]
    mx = jnp.max(o, axis=1, keepdims=True)
    lse = mx + jnp.log(jnp.sum(jnp.exp(o - mx), axis=1, keepdims=True))
    out_ref[:, :] = o - lse


@jax.jit
def kernel(raw_x, adj, W_att, a_att, W_res):
    a_row = a_att.reshape(1, 2 * NCLASS)

    wh, res, src, d2, w2 = pl.pallas_call(
        _prologue_body,
        grid=(NIP,),
        in_specs=[
            pl.BlockSpec((BMP, NFEAT), lambda i: (i, 0)),
            pl.BlockSpec((NFEAT, NCLASS), lambda i: (0, 0)),
            pl.BlockSpec((1, 2 * NCLASS), lambda i: (0, 0)),
            pl.BlockSpec((NFEAT, NCLASS), lambda i: (0, 0)),
        ],
        out_specs=[
            pl.BlockSpec((BMP, NCLASS + 1), lambda i: (i, 0)),
            pl.BlockSpec((BMP, NCLASS), lambda i: (i, 0)),
            pl.BlockSpec((BMP, 1), lambda i: (i, 0)),
            pl.BlockSpec((1, BMP), lambda i: (0, i)),
            pl.BlockSpec((1, BMP), lambda i: (0, i)),
        ],
        out_shape=[
            jax.ShapeDtypeStruct((N, NCLASS + 1), jnp.float32),
            jax.ShapeDtypeStruct((N, NCLASS), jnp.float32),
            jax.ShapeDtypeStruct((N, 1), jnp.float32),
            jax.ShapeDtypeStruct((1, N), jnp.float32),
            jax.ShapeDtypeStruct((1, N), jnp.float32),
        ],
    )(raw_x, W_att, a_row, W_res)

    out = pl.pallas_call(
        _main_body,
        grid=(NI,),
        in_specs=[
            pl.BlockSpec((BM, N), lambda i: (i, 0)),
            pl.BlockSpec((N, NCLASS + 1), lambda i: (0, 0)),
            pl.BlockSpec((BM, 1), lambda i: (i, 0)),
            pl.BlockSpec((1, N), lambda i: (0, 0)),
            pl.BlockSpec((1, N), lambda i: (0, 0)),
            pl.BlockSpec((BM, NCLASS), lambda i: (i, 0)),
        ],
        out_specs=pl.BlockSpec((BM, NCLASS), lambda i: (i, 0)),
        out_shape=jax.ShapeDtypeStruct((N, NCLASS), jnp.float32),
        compiler_params=pltpu.CompilerParams(
            dimension_semantics=("arbitrary",),
        ),
    )(adj, wh, src, d2, w2, res)

    return out


# speedup vs baseline: 2.7526x; 1.0056x over previous
import jax
import jax.numpy as jnp
from jax.experimental import pallas as pl
from jax.experimental.pallas import tpu as pltpu

N = 10000
NFEAT = 128
NCLASS = 16
ALPHA = 0.2
LOG2E = 1.4426950408889634

BM = 128
NI = pl.cdiv(N, BM)
BMP = 512
NIP = pl.cdiv(N, BMP)


def _prologue_body(x_ref, watt_ref, a_ref, wres_ref,
                   wh_ref, res_ref, src_ref, d_ref, w_ref):
    x = x_ref[:, :]
    wh = jnp.dot(x, watt_ref[:, :], preferred_element_type=jnp.float32)
    wh_ref[:, :NCLASS] = wh.astype(jnp.bfloat16)
    wh_ref[:, NCLASS:] = jnp.ones((BMP, 1), jnp.bfloat16)

    res_ref[:, :] = jnp.dot(x, wres_ref[:, :],
                            preferred_element_type=jnp.float32)

    a = a_ref[:, :]
    src_ref[:, :] = jnp.sum(wh * a[:, :NCLASS], axis=1, keepdims=True)

    dst_row = jax.lax.dot_general(
        a[:, NCLASS:], wh, (((1,), (1,)), ((), ())),
        preferred_element_type=jnp.float32)
    d_ref[:, :] = LOG2E * dst_row
    w_ref[:, :] = (ALPHA * LOG2E) * dst_row


def _main_body(adj_ref, wh_ref, src_ref, d_ref, w_ref, res_ref, out_ref):
    dmax2 = jnp.max(d_ref[:, :])
    src2 = LOG2E * src_ref[:, :]
    etop = src2 + dmax2
    c2 = jnp.maximum(etop, ALPHA * etop)
    u = src2 - c2
    v = ALPHA * src2 - c2

    s2 = jnp.maximum(u + d_ref[:, :], v + w_ref[:, :])
    p = (adj_ref[:, :] * jnp.exp2(s2)).astype(jnp.bfloat16)
    acc = jnp.dot(p, wh_ref[:, :], preferred_element_type=jnp.float32)

    h = acc[:, :NCLASS] / acc[:, NCLASS:]
    h = jnp.where(h > 0, h, jnp.exp(h) - 1.0)
    o = h + res_ref[:, :]
    mx = jnp.max(o, axis=1, keepdims=True)
    lse = mx + jnp.log(jnp.sum(jnp.exp(o - mx), axis=1, keepdims=True))
    out_ref[:, :] = o - lse


@jax.jit
def kernel(raw_x, adj, W_att, a_att, W_res):
    a_row = a_att.reshape(1, 2 * NCLASS)

    wh, res, src, d2, w2 = pl.pallas_call(
        _prologue_body,
        grid=(NIP,),
        in_specs=[
            pl.BlockSpec((BMP, NFEAT), lambda i: (i, 0)),
            pl.BlockSpec((NFEAT, NCLASS), lambda i: (0, 0)),
            pl.BlockSpec((1, 2 * NCLASS), lambda i: (0, 0)),
            pl.BlockSpec((NFEAT, NCLASS), lambda i: (0, 0)),
        ],
        out_specs=[
            pl.BlockSpec((BMP, NCLASS + 1), lambda i: (i, 0)),
            pl.BlockSpec((BMP, NCLASS), lambda i: (i, 0)),
            pl.BlockSpec((BMP, 1), lambda i: (i, 0)),
            pl.BlockSpec((1, BMP), lambda i: (0, i)),
            pl.BlockSpec((1, BMP), lambda i: (0, i)),
        ],
        out_shape=[
            jax.ShapeDtypeStruct((N, NCLASS + 1), jnp.bfloat16),
            jax.ShapeDtypeStruct((N, NCLASS), jnp.float32),
            jax.ShapeDtypeStruct((N, 1), jnp.float32),
            jax.ShapeDtypeStruct((1, N), jnp.float32),
            jax.ShapeDtypeStruct((1, N), jnp.float32),
        ],
    )(raw_x, W_att, a_row, W_res)

    out = pl.pallas_call(
        _main_body,
        grid=(NI,),
        in_specs=[
            pl.BlockSpec((BM, N), lambda i: (i, 0)),
            pl.BlockSpec((N, NCLASS + 1), lambda i: (0, 0)),
            pl.BlockSpec((BM, 1), lambda i: (i, 0)),
            pl.BlockSpec((1, N), lambda i: (0, 0)),
            pl.BlockSpec((1, N), lambda i: (0, 0)),
            pl.BlockSpec((BM, NCLASS), lambda i: (i, 0)),
        ],
        out_specs=pl.BlockSpec((BM, NCLASS), lambda i: (i, 0)),
        out_shape=jax.ShapeDtypeStruct((N, NCLASS), jnp.float32),
        compiler_params=pltpu.CompilerParams(
            dimension_semantics=("arbitrary",),
        ),
    )(adj, wh, src, d2, w2, res)

    return out


# speedup vs baseline: 3.2406x; 1.1773x over previous
import jax
import jax.numpy as jnp
from jax.experimental import pallas as pl
from jax.experimental.pallas import tpu as pltpu

N = 10000
NFEAT = 128
NCLASS = 16
ALPHA = 0.2
LOG2E = 1.4426950408889634

BM = 256
NI = pl.cdiv(N, BM)
BMP = 512
NIP = pl.cdiv(N, BMP)


def _prologue_body(x_ref, watt_ref, a_ref, wres_ref,
                   wh_ref, res_ref, src_ref, d_ref, w_ref):
    x = x_ref[:, :]
    wh = jnp.dot(x, watt_ref[:, :], preferred_element_type=jnp.float32)
    wh_ref[:, :NCLASS] = wh.astype(jnp.bfloat16)
    wh_ref[:, NCLASS:] = jnp.ones((BMP, 1), jnp.bfloat16)

    res_ref[:, :] = jnp.dot(x, wres_ref[:, :],
                            preferred_element_type=jnp.float32)

    a = a_ref[:, :]
    src_ref[:, :] = jnp.sum(wh * a[:, :NCLASS], axis=1, keepdims=True)

    dst_row = jax.lax.dot_general(
        a[:, NCLASS:], wh, (((1,), (1,)), ((), ())),
        preferred_element_type=jnp.float32)
    d_ref[:, :] = LOG2E * dst_row
    w_ref[:, :] = (ALPHA * LOG2E) * dst_row


def _main_body(adj_ref, wh_ref, src_ref, d_ref, w_ref, res_ref, out_ref):
    dmax2 = jnp.max(d_ref[:, :])
    src2 = LOG2E * src_ref[:, :]
    etop = src2 + dmax2
    c2 = jnp.maximum(etop, ALPHA * etop)
    u = src2 - c2
    v = ALPHA * src2 - c2

    s2 = jnp.maximum(u + d_ref[:, :], v + w_ref[:, :])
    p = (adj_ref[:, :] * jnp.exp2(s2)).astype(jnp.bfloat16)
    acc = jnp.dot(p, wh_ref[:, :], preferred_element_type=jnp.float32)

    h = acc[:, :NCLASS] / acc[:, NCLASS:]
    h = jnp.where(h > 0, h, jnp.exp(h) - 1.0)
    o = h + res_ref[:, :]
    mx = jnp.max(o, axis=1, keepdims=True)
    lse = mx + jnp.log(jnp.sum(jnp.exp(o - mx), axis=1, keepdims=True))
    out_ref[:, :] = o - lse


@jax.jit
def kernel(raw_x, adj, W_att, a_att, W_res):
    a_row = a_att.reshape(1, 2 * NCLASS)

    wh, res, src, d2, w2 = pl.pallas_call(
        _prologue_body,
        grid=(NIP,),
        in_specs=[
            pl.BlockSpec((BMP, NFEAT), lambda i: (i, 0)),
            pl.BlockSpec((NFEAT, NCLASS), lambda i: (0, 0)),
            pl.BlockSpec((1, 2 * NCLASS), lambda i: (0, 0)),
            pl.BlockSpec((NFEAT, NCLASS), lambda i: (0, 0)),
        ],
        out_specs=[
            pl.BlockSpec((BMP, NCLASS + 1), lambda i: (i, 0)),
            pl.BlockSpec((BMP, NCLASS), lambda i: (i, 0)),
            pl.BlockSpec((BMP, 1), lambda i: (i, 0)),
            pl.BlockSpec((1, BMP), lambda i: (0, i)),
            pl.BlockSpec((1, BMP), lambda i: (0, i)),
        ],
        out_shape=[
            jax.ShapeDtypeStruct((N, NCLASS + 1), jnp.bfloat16),
            jax.ShapeDtypeStruct((N, NCLASS), jnp.float32),
            jax.ShapeDtypeStruct((N, 1), jnp.float32),
            jax.ShapeDtypeStruct((1, N), jnp.float32),
            jax.ShapeDtypeStruct((1, N), jnp.float32),
        ],
    )(raw_x, W_att, a_row, W_res)

    out = pl.pallas_call(
        _main_body,
        grid=(NI,),
        in_specs=[
            pl.BlockSpec((BM, N), lambda i: (i, 0)),
            pl.BlockSpec((N, NCLASS + 1), lambda i: (0, 0)),
            pl.BlockSpec((BM, 1), lambda i: (i, 0)),
            pl.BlockSpec((1, N), lambda i: (0, 0)),
            pl.BlockSpec((1, N), lambda i: (0, 0)),
            pl.BlockSpec((BM, NCLASS), lambda i: (i, 0)),
        ],
        out_specs=pl.BlockSpec((BM, NCLASS), lambda i: (i, 0)),
        out_shape=jax.ShapeDtypeStruct((N, NCLASS), jnp.float32),
        compiler_params=pltpu.CompilerParams(
            dimension_semantics=("arbitrary",),
        ),
    )(adj, wh, src, d2, w2, res)

    return out


# speedup vs baseline: 3.5138x; 1.0843x over previous
import jax
import jax.numpy as jnp
from jax.experimental import pallas as pl
from jax.experimental.pallas import tpu as pltpu

N = 10000
NFEAT = 128
NCLASS = 16
ALPHA = 0.2
LOG2E = 1.4426950408889634

BM = 512
NI = pl.cdiv(N, BM)
BMP = 512
NIP = pl.cdiv(N, BMP)


def _prologue_body(x_ref, watt_ref, a_ref, wres_ref,
                   wh_ref, res_ref, src_ref, d_ref, w_ref):
    x = x_ref[:, :]
    wh = jnp.dot(x, watt_ref[:, :], preferred_element_type=jnp.float32)
    wh_ref[:, :NCLASS] = wh.astype(jnp.bfloat16)
    wh_ref[:, NCLASS:] = jnp.ones((BMP, 1), jnp.bfloat16)

    res_ref[:, :] = jnp.dot(x, wres_ref[:, :],
                            preferred_element_type=jnp.float32)

    a = a_ref[:, :]
    src_ref[:, :] = jnp.sum(wh * a[:, :NCLASS], axis=1, keepdims=True)

    dst_row = jax.lax.dot_general(
        a[:, NCLASS:], wh, (((1,), (1,)), ((), ())),
        preferred_element_type=jnp.float32)
    d_ref[:, :] = LOG2E * dst_row
    w_ref[:, :] = (ALPHA * LOG2E) * dst_row


def _main_body(adj_ref, wh_ref, src_ref, d_ref, w_ref, res_ref, out_ref):
    dmax2 = jnp.max(d_ref[:, :])
    src2 = LOG2E * src_ref[:, :]
    etop = src2 + dmax2
    c2 = jnp.maximum(etop, ALPHA * etop)
    u = src2 - c2
    v = ALPHA * src2 - c2

    s2 = jnp.maximum(u + d_ref[:, :], v + w_ref[:, :])
    p = (adj_ref[:, :] * jnp.exp2(s2)).astype(jnp.bfloat16)
    acc = jnp.dot(p, wh_ref[:, :], preferred_element_type=jnp.float32)

    h = acc[:, :NCLASS] / acc[:, NCLASS:]
    h = jnp.where(h > 0, h, jnp.exp(h) - 1.0)
    o = h + res_ref[:, :]
    mx = jnp.max(o, axis=1, keepdims=True)
    lse = mx + jnp.log(jnp.sum(jnp.exp(o - mx), axis=1, keepdims=True))
    out_ref[:, :] = o - lse


@jax.jit
def kernel(raw_x, adj, W_att, a_att, W_res):
    a_row = a_att.reshape(1, 2 * NCLASS)

    wh, res, src, d2, w2 = pl.pallas_call(
        _prologue_body,
        grid=(NIP,),
        in_specs=[
            pl.BlockSpec((BMP, NFEAT), lambda i: (i, 0)),
            pl.BlockSpec((NFEAT, NCLASS), lambda i: (0, 0)),
            pl.BlockSpec((1, 2 * NCLASS), lambda i: (0, 0)),
            pl.BlockSpec((NFEAT, NCLASS), lambda i: (0, 0)),
        ],
        out_specs=[
            pl.BlockSpec((BMP, NCLASS + 1), lambda i: (i, 0)),
            pl.BlockSpec((BMP, NCLASS), lambda i: (i, 0)),
            pl.BlockSpec((BMP, 1), lambda i: (i, 0)),
            pl.BlockSpec((1, BMP), lambda i: (0, i)),
            pl.BlockSpec((1, BMP), lambda i: (0, i)),
        ],
        out_shape=[
            jax.ShapeDtypeStruct((N, NCLASS + 1), jnp.bfloat16),
            jax.ShapeDtypeStruct((N, NCLASS), jnp.float32),
            jax.ShapeDtypeStruct((N, 1), jnp.float32),
            jax.ShapeDtypeStruct((1, N), jnp.float32),
            jax.ShapeDtypeStruct((1, N), jnp.float32),
        ],
    )(raw_x, W_att, a_row, W_res)

    out = pl.pallas_call(
        _main_body,
        grid=(NI,),
        in_specs=[
            pl.BlockSpec((BM, N), lambda i: (i, 0)),
            pl.BlockSpec((N, NCLASS + 1), lambda i: (0, 0)),
            pl.BlockSpec((BM, 1), lambda i: (i, 0)),
            pl.BlockSpec((1, N), lambda i: (0, 0)),
            pl.BlockSpec((1, N), lambda i: (0, 0)),
            pl.BlockSpec((BM, NCLASS), lambda i: (i, 0)),
        ],
        out_specs=pl.BlockSpec((BM, NCLASS), lambda i: (i, 0)),
        out_shape=jax.ShapeDtypeStruct((N, NCLASS), jnp.float32),
        compiler_params=pltpu.CompilerParams(
            dimension_semantics=("arbitrary",),
        ),
    )(adj, wh, src, d2, w2, res)

    return out
